# baseline (speedup 1.0000x reference)
import functools

import jax
import jax.numpy as jnp
from jax import lax
from jax.experimental import pallas as pl
from jax.experimental.pallas import tpu as pltpu
from jax.experimental.pallas import tpu_sc as plsc

_N = 16384
_M = 16384
_E = 262144
_B = 16

_NC = 2
_NS = 16
_NT = _NC * _NS
_EPT = _E // _NT
_CH = 128
_NCHUNK = _EPT // _CH
_RPT = _N // _NS


def _sc_body(x_hbm, src_hbm, dst_hbm, val_hbm, out_hbm,
             src_v, dst_v, val_v, rows0_v, rows1_v, rows2_v, rows3_v,
             colmaj_v, rowmaj_v, x_sh, acc_sh,
             gsem0, gsem1, gsem2, gsem3, ssem0, ssem1, ssem2, ssem3):
    c = lax.axis_index("c")
    s = lax.axis_index("s")
    wid = c * _NS + s
    rows = (rows0_v, rows1_v, rows2_v, rows3_v)
    gsem = (gsem0, gsem1, gsem2, gsem3)
    ssem = (ssem0, ssem1, ssem2, ssem3)
    lanes = lax.iota(jnp.int32, 16)

    with jax.named_scope("stage_xT"):
        pltpu.sync_copy(x_hbm.at[pl.ds(s * _RPT, _RPT)],
                        x_sh.at[pl.ds(s * _RPT, _RPT)])

        def _zero_row(i, _):
            rows0_v[i, :] = jnp.zeros((_B,), jnp.float32)
            return 0
        lax.fori_loop(0, _CH, _zero_row, 0, unroll=8)
        def _zero_acc(k, _):
            pltpu.sync_copy(rows0_v, acc_sh.at[pl.ds(s * _RPT + k * _CH, _CH)])
            return 0
        lax.fori_loop(0, _RPT // _CH, _zero_acc, 0)

        base = wid * _NCHUNK
        pltpu.sync_copy(src_hbm.at[pl.ds(base, _NCHUNK)], src_v)
        pltpu.sync_copy(dst_hbm.at[pl.ds(base, _NCHUNK)], dst_v)
        pltpu.sync_copy(val_hbm.at[pl.ds(base, _NCHUNK)], val_v)

        plsc.subcore_barrier()

    def _gather(j, b):
        jc = jnp.minimum(j, _NCHUNK - 1)
        pltpu.async_copy(x_sh.at[src_v.at[jc]], rows[b], gsem[b])

    def _wait_gather(b):
        pltpu.make_async_copy(x_sh.at[src_v.at[0]], rows[b], gsem[b]).wait()

    def _wait_scatter(b):
        pltpu.make_async_copy(rows[b], acc_sh.at[dst_v.at[0]], ssem[b]).wait()

    def _process(j, b):
        _wait_gather(b)
        rv = rows[b]

        @plsc.parallel_loop(0, _CH // 16, unroll=2)
        def _mulgrp(g):
            vv = val_v[j, pl.ds(g * 16, 16)]
            for e in range(16):
                bf = lax.gather(
                    vv, jnp.full((16, 1), e, jnp.int32),
                    lax.GatherDimensionNumbers(offset_dims=(),
                                               collapsed_slice_dims=(0,),
                                               start_index_map=(0,)),
                    (1,), mode=lax.GatherScatterMode.PROMISE_IN_BOUNDS)
                i = g * 16 + e
                rv[i, :] = rv[i, :] * bf
        pltpu.async_copy(rv, acc_sh.at[dst_v.at[j]], ssem[b], add=True)

    with jax.named_scope("edge_loop"):
        _gather(jnp.int32(0), 0)
        _gather(jnp.int32(1), 1)
        _gather(jnp.int32(2), 2)

        _process(jnp.int32(0), 0)
        _gather(jnp.int32(3), 3)
        for bi in range(1, 4):
            _process(jnp.int32(bi), bi)
            _wait_scatter(bi - 1)
            _gather(jnp.int32(bi + 3), bi - 1)

        def _group(j2, _):
            j = j2 * 4
            for bi in range(4):
                _process(j + bi, bi)
                bpre = (bi + 3) % 4
                _wait_scatter(bpre)
                _gather(j + bi + 3, bpre)
            return 0
        lax.fori_loop(1, _NCHUNK // 4, _group, 0)
        _wait_gather(0)
        _wait_gather(1)
        _wait_gather(2)
        _wait_scatter(3)

        plsc.subcore_barrier()

    with jax.named_scope("emit_out"):
        pltpu.sync_copy(acc_sh.at[pl.ds(s * _RPT, _RPT)], colmaj_v)

        @plsc.parallel_loop(0, _RPT, unroll=8)
        def _txpo(i):
            row = colmaj_v[i, :]
            plsc.store_scatter(rowmaj_v,
                               [lanes, jnp.full((16,), i, jnp.int32)], row)
        pltpu.sync_copy(rowmaj_v, out_hbm.at[c, :, pl.ds(s * _RPT, _RPT)])


_sc_call = functools.partial(
    pl.kernel,
    out_type=jax.ShapeDtypeStruct((_NC, _B, _M), jnp.float32),
    mesh=plsc.VectorSubcoreMesh(core_axis_name="c", subcore_axis_name="s"),
    scratch_types=[
        pltpu.VMEM((_NCHUNK, _CH), jnp.int32),
        pltpu.VMEM((_NCHUNK, _CH), jnp.int32),
        pltpu.VMEM((_NCHUNK, _CH), jnp.float32),
        pltpu.VMEM((_CH, _B), jnp.float32),
        pltpu.VMEM((_CH, _B), jnp.float32),
        pltpu.VMEM((_CH, _B), jnp.float32),
        pltpu.VMEM((_CH, _B), jnp.float32),
        pltpu.VMEM((_RPT, _B), jnp.float32),
        pltpu.VMEM((_B, _RPT), jnp.float32),
        pltpu.VMEM_SHARED((_N, _B), jnp.float32),
        pltpu.VMEM_SHARED((_M, _B), jnp.float32),
        pltpu.SemaphoreType.DMA,
        pltpu.SemaphoreType.DMA,
        pltpu.SemaphoreType.DMA,
        pltpu.SemaphoreType.DMA,
        pltpu.SemaphoreType.DMA,
        pltpu.SemaphoreType.DMA,
        pltpu.SemaphoreType.DMA,
        pltpu.SemaphoreType.DMA,
    ],
    compiler_params=pltpu.CompilerParams(use_tc_tiling_on_sc=False,
                                         needs_layout_passes=False),
)(_sc_body)


def _txp_body(x_ref, o_ref):
    o_ref[...] = jnp.transpose(x_ref[...])


def _tc_transpose(x2):
    blk = 2048
    return pl.pallas_call(
        _txp_body,
        grid=(_N // blk,),
        in_specs=[pl.BlockSpec((_B, blk), lambda i: (0, i))],
        out_specs=pl.BlockSpec((blk, _B), lambda i: (i, 0)),
        out_shape=jax.ShapeDtypeStruct((_N, _B), jnp.float32),
    )(x2)


def _combine_body(p_ref, b_ref, o_ref):
    o_ref[...] = p_ref[0] + p_ref[1] + b_ref[...]


def _combine(partials, bias_row):
    return pl.pallas_call(
        _combine_body,
        in_specs=[
            pl.BlockSpec((_NC, _B, _M), lambda: (0, 0, 0)),
            pl.BlockSpec((1, _M), lambda: (0, 0)),
        ],
        out_specs=pl.BlockSpec((_B, _M), lambda: (0, 0)),
        out_shape=jax.ShapeDtypeStruct((_B, _M), jnp.float32),
    )(partials, bias_row)


def kernel(x, values, bias, edge_index):
    xT = _tc_transpose(x.reshape(_B, _N))
    src2 = edge_index[0].reshape(_E // _CH, _CH)
    dst2 = edge_index[1].reshape(_E // _CH, _CH)
    val2 = values.reshape(_E // _CH, _CH)
    partials = _sc_call(xT, src2, dst2, val2)
    combined = _combine(partials, bias.reshape(1, _M))
    return combined[:, :, None]


# speedup vs baseline: 1.1136x; 1.1136x over previous
import functools

import jax
import jax.numpy as jnp
from jax import lax
from jax.experimental import pallas as pl
from jax.experimental.pallas import tpu as pltpu
from jax.experimental.pallas import tpu_sc as plsc

_N = 16384
_M = 16384
_E = 262144
_B = 16

_NC = 2
_NS = 16
_NT = _NC * _NS
_EPT = _E // _NT
_CH = 128
_NCHUNK = _EPT // _CH
_RPT = _N // _NS


def _sc_body(x_hbm, src_hbm, dst_hbm, val_hbm, out_hbm,
             src_v, dst_v, val_v, rows0_v, rows1_v, rows2_v, rows3_v,
             colmaj_v, rowmaj_v, x_sh, acc_sh,
             gsem0, gsem1, gsem2, gsem3, ssem0, ssem1, ssem2, ssem3):
    c = lax.axis_index("c")
    s = lax.axis_index("s")
    wid = c * _NS + s
    rows = (rows0_v, rows1_v, rows2_v, rows3_v)
    gsem = (gsem0, gsem1, gsem2, gsem3)
    ssem = (ssem0, ssem1, ssem2, ssem3)
    lanes = lax.iota(jnp.int32, 16)

    with jax.named_scope("stage_xT"):
        pltpu.sync_copy(x_hbm.at[:, pl.ds(s * _RPT, _RPT)], rowmaj_v)

        @plsc.parallel_loop(0, _RPT, unroll=16)
        def _txp(i):
            col = plsc.load_gather(rowmaj_v,
                                   [lanes, jnp.full((16,), i, jnp.int32)])
            colmaj_v[i, :] = col
        pltpu.sync_copy(colmaj_v, x_sh.at[pl.ds(s * _RPT, _RPT)])

        def _zero_row(i, _):
            rows0_v[i, :] = jnp.zeros((_B,), jnp.float32)
            return 0
        lax.fori_loop(0, _CH, _zero_row, 0, unroll=8)
        def _zero_acc(k, _):
            pltpu.sync_copy(rows0_v, acc_sh.at[pl.ds(s * _RPT + k * _CH, _CH)])
            return 0
        lax.fori_loop(0, _RPT // _CH, _zero_acc, 0)

        base = wid * _NCHUNK
        pltpu.sync_copy(src_hbm.at[pl.ds(base, _NCHUNK)], src_v)
        pltpu.sync_copy(dst_hbm.at[pl.ds(base, _NCHUNK)], dst_v)
        pltpu.sync_copy(val_hbm.at[pl.ds(base, _NCHUNK)], val_v)

        plsc.subcore_barrier()

    def _gather(j, b):
        jc = jnp.minimum(j, _NCHUNK - 1)
        pltpu.async_copy(x_sh.at[src_v.at[jc]], rows[b], gsem[b])

    def _wait_gather(b):
        pltpu.make_async_copy(x_sh.at[src_v.at[0]], rows[b], gsem[b]).wait()

    def _wait_scatter(b):
        pltpu.make_async_copy(rows[b], acc_sh.at[dst_v.at[0]], ssem[b]).wait()

    def _process(j, b):
        _wait_gather(b)
        rv = rows[b]

        @plsc.parallel_loop(0, _CH // 16, unroll=4)
        def _mulgrp(g):
            vv = val_v[j, pl.ds(g * 16, 16)]
            for e in range(16):
                bf = lax.gather(
                    vv, jnp.full((16, 1), e, jnp.int32),
                    lax.GatherDimensionNumbers(offset_dims=(),
                                               collapsed_slice_dims=(0,),
                                               start_index_map=(0,)),
                    (1,), mode=lax.GatherScatterMode.PROMISE_IN_BOUNDS)
                i = g * 16 + e
                rv[i, :] = rv[i, :] * bf
        pltpu.async_copy(rv, acc_sh.at[dst_v.at[j]], ssem[b], add=True)

    with jax.named_scope("edge_loop"):
        _gather(jnp.int32(0), 0)
        _gather(jnp.int32(1), 1)
        _gather(jnp.int32(2), 2)

        _process(jnp.int32(0), 0)
        _gather(jnp.int32(3), 3)
        for bi in range(1, 4):
            _process(jnp.int32(bi), bi)
            _wait_scatter(bi - 1)
            _gather(jnp.int32(bi + 3), bi - 1)

        def _group(j2, _):
            j = j2 * 4
            for bi in range(4):
                _process(j + bi, bi)
                bpre = (bi + 3) % 4
                _wait_scatter(bpre)
                _gather(j + bi + 3, bpre)
            return 0
        lax.fori_loop(1, _NCHUNK // 4, _group, 0)
        _wait_gather(0)
        _wait_gather(1)
        _wait_gather(2)
        _wait_scatter(3)

        plsc.subcore_barrier()

    with jax.named_scope("emit_out"):
        pltpu.sync_copy(acc_sh.at[pl.ds(s * _RPT, _RPT)], colmaj_v)

        @plsc.parallel_loop(0, _RPT, unroll=16)
        def _txpo(i):
            row = colmaj_v[i, :]
            plsc.store_scatter(rowmaj_v,
                               [lanes, jnp.full((16,), i, jnp.int32)], row)
        pltpu.sync_copy(rowmaj_v, out_hbm.at[c, :, pl.ds(s * _RPT, _RPT)])


_sc_call = functools.partial(
    pl.kernel,
    out_type=jax.ShapeDtypeStruct((_NC, _B, _M), jnp.float32),
    mesh=plsc.VectorSubcoreMesh(core_axis_name="c", subcore_axis_name="s"),
    scratch_types=[
        pltpu.VMEM((_NCHUNK, _CH), jnp.int32),
        pltpu.VMEM((_NCHUNK, _CH), jnp.int32),
        pltpu.VMEM((_NCHUNK, _CH), jnp.float32),
        pltpu.VMEM((_CH, _B), jnp.float32),
        pltpu.VMEM((_CH, _B), jnp.float32),
        pltpu.VMEM((_CH, _B), jnp.float32),
        pltpu.VMEM((_CH, _B), jnp.float32),
        pltpu.VMEM((_RPT, _B), jnp.float32),
        pltpu.VMEM((_B, _RPT), jnp.float32),
        pltpu.VMEM_SHARED((_N, _B), jnp.float32),
        pltpu.VMEM_SHARED((_M, _B), jnp.float32),
        pltpu.SemaphoreType.DMA,
        pltpu.SemaphoreType.DMA,
        pltpu.SemaphoreType.DMA,
        pltpu.SemaphoreType.DMA,
        pltpu.SemaphoreType.DMA,
        pltpu.SemaphoreType.DMA,
        pltpu.SemaphoreType.DMA,
        pltpu.SemaphoreType.DMA,
    ],
    compiler_params=pltpu.CompilerParams(use_tc_tiling_on_sc=False,
                                         needs_layout_passes=False),
)(_sc_body)


def _combine_body(p_ref, b_ref, o_ref):
    o_ref[...] = p_ref[0] + p_ref[1] + b_ref[...]


def _combine(partials, bias_row):
    return pl.pallas_call(
        _combine_body,
        in_specs=[
            pl.BlockSpec((_NC, _B, _M), lambda: (0, 0, 0)),
            pl.BlockSpec((1, _M), lambda: (0, 0)),
        ],
        out_specs=pl.BlockSpec((_B, _M), lambda: (0, 0)),
        out_shape=jax.ShapeDtypeStruct((_B, _M), jnp.float32),
    )(partials, bias_row)


def kernel(x, values, bias, edge_index):
    x2 = x.reshape(_B, _N)
    src2 = edge_index[0].reshape(_E // _CH, _CH)
    dst2 = edge_index[1].reshape(_E // _CH, _CH)
    val2 = values.reshape(_E // _CH, _CH)
    partials = _sc_call(x2, src2, dst2, val2)
    combined = _combine(partials, bias.reshape(1, _M))
    return combined[:, :, None]


# speedup vs baseline: 1.1646x; 1.0459x over previous
import functools

import jax
import jax.numpy as jnp
from jax import lax
from jax.experimental import pallas as pl
from jax.experimental.pallas import tpu as pltpu
from jax.experimental.pallas import tpu_sc as plsc

_N = 16384
_M = 16384
_E = 262144
_B = 16

_NC = 2
_NS = 16
_NT = _NC * _NS
_EPT = _E // _NT
_CH = 128
_NCHUNK = _EPT // _CH
_RPT = _N // _NS


def _sc_body(x_hbm, src_hbm, dst_hbm, val_hbm, out_hbm,
             src_v, dst_v, val_v, rows0_v, rows1_v, rows2_v, rows3_v,
             colmaj_v, rowmaj_v, x_sh, acc_sh,
             gsem0, gsem1, gsem2, gsem3, ssem0, ssem1, ssem2, ssem3):
    c = lax.axis_index("c")
    s = lax.axis_index("s")
    wid = c * _NS + s
    rows = (rows0_v, rows1_v, rows2_v, rows3_v)
    gsem = (gsem0, gsem1, gsem2, gsem3)
    ssem = (ssem0, ssem1, ssem2, ssem3)
    lanes = lax.iota(jnp.int32, 16)

    with jax.named_scope("stage_xT"):
        base = wid * _NCHUNK
        cp_x = pltpu.async_copy(x_hbm.at[:, pl.ds(s * _RPT, _RPT)],
                                rowmaj_v, gsem[0])
        cp_s = pltpu.async_copy(src_hbm.at[pl.ds(base, _NCHUNK)], src_v,
                                gsem[1])
        cp_d = pltpu.async_copy(dst_hbm.at[pl.ds(base, _NCHUNK)], dst_v,
                                gsem[2])
        cp_v = pltpu.async_copy(val_hbm.at[pl.ds(base, _NCHUNK)], val_v,
                                gsem[3])

        @plsc.parallel_loop(0, _CH, unroll=8)
        def _zero_row(i):
            rows0_v[i, :] = jnp.zeros((_B,), jnp.float32)

        for k in range(_RPT // _CH):
            pltpu.async_copy(rows0_v,
                             acc_sh.at[pl.ds(s * _RPT + k * _CH, _CH)],
                             ssem[0])

        cp_x.wait()

        @plsc.parallel_loop(0, _RPT, unroll=16)
        def _txp(i):
            col = plsc.load_gather(rowmaj_v,
                                   [lanes, jnp.full((16,), i, jnp.int32)])
            colmaj_v[i, :] = col
        pltpu.sync_copy(colmaj_v, x_sh.at[pl.ds(s * _RPT, _RPT)])

        cp_s.wait()
        cp_d.wait()
        cp_v.wait()
        for k in range(_RPT // _CH):
            pltpu.make_async_copy(rows0_v,
                                  acc_sh.at[pl.ds(s * _RPT, _CH)],
                                  ssem[0]).wait()

        plsc.subcore_barrier()

    def _gather(j, b):
        jc = jnp.minimum(j, _NCHUNK - 1)
        pltpu.async_copy(x_sh.at[src_v.at[jc]], rows[b], gsem[b])

    def _wait_gather(b):
        pltpu.make_async_copy(x_sh.at[src_v.at[0]], rows[b], gsem[b]).wait()

    def _wait_scatter(b):
        pltpu.make_async_copy(rows[b], acc_sh.at[dst_v.at[0]], ssem[b]).wait()

    def _process(j, b):
        _wait_gather(b)
        rv = rows[b]

        @plsc.parallel_loop(0, _CH // 16, unroll=4)
        def _mulgrp(g):
            vv = val_v[j, pl.ds(g * 16, 16)]
            for e in range(16):
                bf = lax.gather(
                    vv, jnp.full((16, 1), e, jnp.int32),
                    lax.GatherDimensionNumbers(offset_dims=(),
                                               collapsed_slice_dims=(0,),
                                               start_index_map=(0,)),
                    (1,), mode=lax.GatherScatterMode.PROMISE_IN_BOUNDS)
                i = g * 16 + e
                rv[i, :] = rv[i, :] * bf
        pltpu.async_copy(rv, acc_sh.at[dst_v.at[j]], ssem[b], add=True)

    with jax.named_scope("edge_loop"):
        _gather(jnp.int32(0), 0)
        _gather(jnp.int32(1), 1)
        _gather(jnp.int32(2), 2)

        _process(jnp.int32(0), 0)
        _gather(jnp.int32(3), 3)
        for bi in range(1, 4):
            _process(jnp.int32(bi), bi)
            _wait_scatter(bi - 1)
            _gather(jnp.int32(bi + 3), bi - 1)

        def _group(j2, _):
            j = j2 * 4
            for bi in range(4):
                _process(j + bi, bi)
                bpre = (bi + 3) % 4
                _wait_scatter(bpre)
                _gather(j + bi + 3, bpre)
            return 0
        lax.fori_loop(1, _NCHUNK // 4, _group, 0)
        _wait_gather(0)
        _wait_gather(1)
        _wait_gather(2)
        _wait_scatter(3)

        plsc.subcore_barrier()

    with jax.named_scope("emit_out"):
        pltpu.sync_copy(acc_sh.at[pl.ds(s * _RPT, _RPT)], colmaj_v)

        @plsc.parallel_loop(0, _RPT, unroll=16)
        def _txpo(i):
            row = colmaj_v[i, :]
            plsc.store_scatter(rowmaj_v,
                               [lanes, jnp.full((16,), i, jnp.int32)], row)
        pltpu.sync_copy(rowmaj_v, out_hbm.at[c, :, pl.ds(s * _RPT, _RPT)])


_sc_call = functools.partial(
    pl.kernel,
    out_type=jax.ShapeDtypeStruct((_NC, _B, _M), jnp.float32),
    mesh=plsc.VectorSubcoreMesh(core_axis_name="c", subcore_axis_name="s"),
    scratch_types=[
        pltpu.VMEM((_NCHUNK, _CH), jnp.int32),
        pltpu.VMEM((_NCHUNK, _CH), jnp.int32),
        pltpu.VMEM((_NCHUNK, _CH), jnp.float32),
        pltpu.VMEM((_CH, _B), jnp.float32),
        pltpu.VMEM((_CH, _B), jnp.float32),
        pltpu.VMEM((_CH, _B), jnp.float32),
        pltpu.VMEM((_CH, _B), jnp.float32),
        pltpu.VMEM((_RPT, _B), jnp.float32),
        pltpu.VMEM((_B, _RPT), jnp.float32),
        pltpu.VMEM_SHARED((_N, _B), jnp.float32),
        pltpu.VMEM_SHARED((_M, _B), jnp.float32),
        pltpu.SemaphoreType.DMA,
        pltpu.SemaphoreType.DMA,
        pltpu.SemaphoreType.DMA,
        pltpu.SemaphoreType.DMA,
        pltpu.SemaphoreType.DMA,
        pltpu.SemaphoreType.DMA,
        pltpu.SemaphoreType.DMA,
        pltpu.SemaphoreType.DMA,
    ],
    compiler_params=pltpu.CompilerParams(use_tc_tiling_on_sc=False,
                                         needs_layout_passes=False),
)(_sc_body)


def _combine_body(p_ref, b_ref, o_ref):
    o_ref[...] = p_ref[0] + p_ref[1] + b_ref[...]


def _combine(partials, bias_row):
    return pl.pallas_call(
        _combine_body,
        in_specs=[
            pl.BlockSpec((_NC, _B, _M), lambda: (0, 0, 0)),
            pl.BlockSpec((1, _M), lambda: (0, 0)),
        ],
        out_specs=pl.BlockSpec((_B, _M), lambda: (0, 0)),
        out_shape=jax.ShapeDtypeStruct((_B, _M), jnp.float32),
    )(partials, bias_row)


def kernel(x, values, bias, edge_index):
    x2 = x.reshape(_B, _N)
    src2 = edge_index[0].reshape(_E // _CH, _CH)
    dst2 = edge_index[1].reshape(_E // _CH, _CH)
    val2 = values.reshape(_E // _CH, _CH)
    partials = _sc_call(x2, src2, dst2, val2)
    combined = _combine(partials, bias.reshape(1, _M))
    return combined[:, :, None]


# speedup vs baseline: 1.1735x; 1.0076x over previous
import functools

import jax
import jax.numpy as jnp
from jax import lax
from jax.experimental import pallas as pl
from jax.experimental.pallas import tpu as pltpu
from jax.experimental.pallas import tpu_sc as plsc

_N = 16384
_M = 16384
_E = 262144
_B = 16

_NC = 2
_NS = 16
_NT = _NC * _NS
_EPT = _E // _NT
_CH = 128
_NCHUNK = _EPT // _CH
_RPT = _N // _NS


def _sc_body(x_hbm, src_hbm, dst_hbm, val_hbm, out_hbm,
             src_v, dst_v, val_v, rows0_v, rows1_v, rows2_v, rows3_v,
             colmaj_v, rowmaj_v, x_sh, acc_sh,
             gsem0, gsem1, gsem2, gsem3, ssem0, ssem1, ssem2, ssem3):
    c = lax.axis_index("c")
    s = lax.axis_index("s")
    wid = c * _NS + s
    rows = (rows0_v, rows1_v, rows2_v, rows3_v)
    gsem = (gsem0, gsem1, gsem2, gsem3)
    ssem = (ssem0, ssem1, ssem2, ssem3)
    lanes = lax.iota(jnp.int32, 16)

    with jax.named_scope("stage_xT"):
        base = wid * _NCHUNK
        cp_x = pltpu.async_copy(x_hbm.at[:, pl.ds(s * _RPT, _RPT)],
                                rowmaj_v, gsem[0])
        cp_s = pltpu.async_copy(src_hbm.at[pl.ds(base, _NCHUNK)], src_v,
                                gsem[1])
        cp_d = pltpu.async_copy(dst_hbm.at[pl.ds(base, _NCHUNK)], dst_v,
                                gsem[2])
        cp_v = pltpu.async_copy(val_hbm.at[pl.ds(base, _NCHUNK)], val_v,
                                gsem[3])

        @plsc.parallel_loop(0, _CH, unroll=8)
        def _zero_row(i):
            rows0_v[i, :] = jnp.zeros((_B,), jnp.float32)

        for k in range(_RPT // _CH):
            pltpu.async_copy(rows0_v,
                             acc_sh.at[pl.ds(s * _RPT + k * _CH, _CH)],
                             ssem[0])

        cp_x.wait()

        @plsc.parallel_loop(0, _RPT, unroll=16)
        def _txp(i):
            col = plsc.load_gather(rowmaj_v,
                                   [lanes, jnp.full((16,), i, jnp.int32)])
            colmaj_v[i, :] = col
        pltpu.sync_copy(colmaj_v, x_sh.at[pl.ds(s * _RPT, _RPT)])

        cp_s.wait()
        cp_d.wait()
        cp_v.wait()
        for k in range(_RPT // _CH):
            pltpu.make_async_copy(rows0_v,
                                  acc_sh.at[pl.ds(s * _RPT, _CH)],
                                  ssem[0]).wait()

        plsc.subcore_barrier()

    def _gather(j, b):
        jc = jnp.minimum(j, _NCHUNK - 1)
        pltpu.async_copy(x_sh.at[src_v.at[jc]], rows[b], gsem[b])

    def _wait_gather(b):
        pltpu.make_async_copy(x_sh.at[src_v.at[0]], rows[b], gsem[b]).wait()

    def _wait_scatter(b):
        pltpu.make_async_copy(rows[b], acc_sh.at[dst_v.at[0]], ssem[b]).wait()

    def _process(j, b):
        _wait_gather(b)
        rv = rows[b]

        @plsc.parallel_loop(0, _CH // 16, unroll=4)
        def _mulgrp(g):
            vv = val_v[j, pl.ds(g * 16, 16)]
            for e in range(16):
                bf = lax.gather(
                    vv, jnp.full((16, 1), e, jnp.int32),
                    lax.GatherDimensionNumbers(offset_dims=(),
                                               collapsed_slice_dims=(0,),
                                               start_index_map=(0,)),
                    (1,), mode=lax.GatherScatterMode.PROMISE_IN_BOUNDS)
                i = g * 16 + e
                rv[i, :] = rv[i, :] * bf
        pltpu.async_copy(rv, acc_sh.at[dst_v.at[j]], ssem[b], add=True)

    with jax.named_scope("edge_loop"):
        _gather(jnp.int32(0), 0)
        _gather(jnp.int32(1), 1)
        _gather(jnp.int32(2), 2)

        _process(jnp.int32(0), 0)
        _gather(jnp.int32(3), 3)
        for bi in range(1, 4):
            _process(jnp.int32(bi), bi)
            _wait_scatter(bi - 1)
            _gather(jnp.int32(bi + 3), bi - 1)

        def _group(j2, _):
            j = j2 * 4
            for bi in range(4):
                _process(j + bi, bi)
                bpre = (bi + 3) % 4
                _wait_scatter(bpre)
                _gather(j + bi + 3, bpre)
            return 0
        lax.fori_loop(1, _NCHUNK // 4, _group, 0)
        _wait_gather(0)
        _wait_gather(1)
        _wait_gather(2)
        _wait_scatter(3)

        plsc.subcore_barrier()

    with jax.named_scope("emit_out"):
        _QR = _RPT // 4
        for k in range(4):
            pltpu.async_copy(acc_sh.at[pl.ds(s * _RPT + k * _QR, _QR)],
                             colmaj_v.at[pl.ds(k * _QR, _QR)], gsem[k])
        for k in range(4):
            pltpu.make_async_copy(acc_sh.at[pl.ds(s * _RPT, _QR)],
                                  colmaj_v.at[pl.ds(k * _QR, _QR)],
                                  gsem[k]).wait()

            @plsc.parallel_loop(k * _QR, (k + 1) * _QR, unroll=16)
            def _txpo(i):
                row = colmaj_v[i, :]
                plsc.store_scatter(rowmaj_v,
                                   [lanes, jnp.full((16,), i, jnp.int32)], row)
            pltpu.async_copy(rowmaj_v.at[:, pl.ds(k * _QR, _QR)],
                             out_hbm.at[c, :, pl.ds(s * _RPT + k * _QR, _QR)],
                             ssem[0])
        for k in range(4):
            pltpu.make_async_copy(rowmaj_v.at[:, pl.ds(0, _QR)],
                                  out_hbm.at[c, :, pl.ds(s * _RPT, _QR)],
                                  ssem[0]).wait()


_sc_call = functools.partial(
    pl.kernel,
    out_type=jax.ShapeDtypeStruct((_NC, _B, _M), jnp.float32),
    mesh=plsc.VectorSubcoreMesh(core_axis_name="c", subcore_axis_name="s"),
    scratch_types=[
        pltpu.VMEM((_NCHUNK, _CH), jnp.int32),
        pltpu.VMEM((_NCHUNK, _CH), jnp.int32),
        pltpu.VMEM((_NCHUNK, _CH), jnp.float32),
        pltpu.VMEM((_CH, _B), jnp.float32),
        pltpu.VMEM((_CH, _B), jnp.float32),
        pltpu.VMEM((_CH, _B), jnp.float32),
        pltpu.VMEM((_CH, _B), jnp.float32),
        pltpu.VMEM((_RPT, _B), jnp.float32),
        pltpu.VMEM((_B, _RPT), jnp.float32),
        pltpu.VMEM_SHARED((_N, _B), jnp.float32),
        pltpu.VMEM_SHARED((_M, _B), jnp.float32),
        pltpu.SemaphoreType.DMA,
        pltpu.SemaphoreType.DMA,
        pltpu.SemaphoreType.DMA,
        pltpu.SemaphoreType.DMA,
        pltpu.SemaphoreType.DMA,
        pltpu.SemaphoreType.DMA,
        pltpu.SemaphoreType.DMA,
        pltpu.SemaphoreType.DMA,
    ],
    compiler_params=pltpu.CompilerParams(use_tc_tiling_on_sc=False,
                                         needs_layout_passes=False),
)(_sc_body)


def _combine_body(p_ref, b_ref, o_ref):
    o_ref[...] = p_ref[0] + p_ref[1] + b_ref[...]


def _combine(partials, bias_row):
    return pl.pallas_call(
        _combine_body,
        in_specs=[
            pl.BlockSpec((_NC, _B, _M), lambda: (0, 0, 0)),
            pl.BlockSpec((1, _M), lambda: (0, 0)),
        ],
        out_specs=pl.BlockSpec((_B, _M), lambda: (0, 0)),
        out_shape=jax.ShapeDtypeStruct((_B, _M), jnp.float32),
    )(partials, bias_row)


def kernel(x, values, bias, edge_index):
    x2 = x.reshape(_B, _N)
    src2 = edge_index[0].reshape(_E // _CH, _CH)
    dst2 = edge_index[1].reshape(_E // _CH, _CH)
    val2 = values.reshape(_E // _CH, _CH)
    partials = _sc_call(x2, src2, dst2, val2)
    combined = _combine(partials, bias.reshape(1, _M))
    return combined[:, :, None]


# speedup vs baseline: 1.1772x; 1.0032x over previous
import functools

import jax
import jax.numpy as jnp
from jax import lax
from jax.experimental import pallas as pl
from jax.experimental.pallas import tpu as pltpu
from jax.experimental.pallas import tpu_sc as plsc

_N = 16384
_M = 16384
_E = 262144
_B = 16

_NC = 2
_NS = 16
_NT = _NC * _NS
_EPT = _E // _NT
_CH = 128
_NCHUNK = _EPT // _CH
_RPT = _N // _NS


def _sc_body(x_hbm, src_hbm, dst_hbm, val_hbm, out_hbm,
             src_v, dst_v, val_v, rows0_v, rows1_v, rows2_v, rows3_v,
             colmaj_v, rowmaj_v, x_sh, acc_sh,
             gsem0, gsem1, gsem2, gsem3, ssem0, ssem1, ssem2, ssem3):
    c = lax.axis_index("c")
    s = lax.axis_index("s")
    wid = c * _NS + s
    rows = (rows0_v, rows1_v, rows2_v, rows3_v)
    gsem = (gsem0, gsem1, gsem2, gsem3)
    ssem = (ssem0, ssem1, ssem2, ssem3)
    lanes = lax.iota(jnp.int32, 16)

    with jax.named_scope("stage_xT"):
        base = wid * _NCHUNK
        cp_x = pltpu.async_copy(x_hbm.at[:, pl.ds(s * _RPT, _RPT)],
                                rowmaj_v, gsem[0])
        cp_s = pltpu.async_copy(src_hbm.at[pl.ds(base, _NCHUNK)], src_v,
                                gsem[1])
        cp_d = pltpu.async_copy(dst_hbm.at[pl.ds(base, _NCHUNK)], dst_v,
                                gsem[2])
        cp_v = pltpu.async_copy(val_hbm.at[pl.ds(base, _NCHUNK)], val_v,
                                gsem[3])

        @plsc.parallel_loop(0, _CH, unroll=8)
        def _zero_row(i):
            rows0_v[i, :] = jnp.zeros((_B,), jnp.float32)

        for k in range(_RPT // _CH):
            pltpu.async_copy(rows0_v,
                             acc_sh.at[pl.ds(s * _RPT + k * _CH, _CH)],
                             ssem[0])

        cp_x.wait()

        _QS = _RPT // 4
        for k in range(4):
            @plsc.parallel_loop(k * _QS, (k + 1) * _QS, unroll=16)
            def _txp(i):
                col = plsc.load_gather(rowmaj_v,
                                       [lanes, jnp.full((16,), i, jnp.int32)])
                colmaj_v[i, :] = col
            pltpu.async_copy(colmaj_v.at[pl.ds(k * _QS, _QS)],
                             x_sh.at[pl.ds(s * _RPT + k * _QS, _QS)], gsem[0])
        for k in range(4):
            pltpu.make_async_copy(colmaj_v.at[pl.ds(0, _QS)],
                                  x_sh.at[pl.ds(s * _RPT, _QS)],
                                  gsem[0]).wait()

        cp_s.wait()
        cp_d.wait()
        cp_v.wait()
        for k in range(_RPT // _CH):
            pltpu.make_async_copy(rows0_v,
                                  acc_sh.at[pl.ds(s * _RPT, _CH)],
                                  ssem[0]).wait()

        plsc.subcore_barrier()

    def _gather(j, b):
        jc = jnp.minimum(j, _NCHUNK - 1)
        pltpu.async_copy(x_sh.at[src_v.at[jc]], rows[b], gsem[b])

    def _wait_gather(b):
        pltpu.make_async_copy(x_sh.at[src_v.at[0]], rows[b], gsem[b]).wait()

    def _wait_scatter(b):
        pltpu.make_async_copy(rows[b], acc_sh.at[dst_v.at[0]], ssem[b]).wait()

    def _process(j, b):
        _wait_gather(b)
        rv = rows[b]

        @plsc.parallel_loop(0, _CH // 16, unroll=4)
        def _mulgrp(g):
            vv = val_v[j, pl.ds(g * 16, 16)]
            for e in range(16):
                bf = lax.gather(
                    vv, jnp.full((16, 1), e, jnp.int32),
                    lax.GatherDimensionNumbers(offset_dims=(),
                                               collapsed_slice_dims=(0,),
                                               start_index_map=(0,)),
                    (1,), mode=lax.GatherScatterMode.PROMISE_IN_BOUNDS)
                i = g * 16 + e
                rv[i, :] = rv[i, :] * bf
        pltpu.async_copy(rv, acc_sh.at[dst_v.at[j]], ssem[b], add=True)

    with jax.named_scope("edge_loop"):
        _gather(jnp.int32(0), 0)
        _gather(jnp.int32(1), 1)
        _gather(jnp.int32(2), 2)

        _process(jnp.int32(0), 0)
        _gather(jnp.int32(3), 3)
        for bi in range(1, 4):
            _process(jnp.int32(bi), bi)
            _wait_scatter(bi - 1)
            _gather(jnp.int32(bi + 3), bi - 1)

        def _group(j2, _):
            j = j2 * 4
            for bi in range(4):
                _process(j + bi, bi)
                bpre = (bi + 3) % 4
                _wait_scatter(bpre)
                _gather(j + bi + 3, bpre)
            return 0
        lax.fori_loop(1, _NCHUNK // 4, _group, 0)
        _wait_gather(0)
        _wait_gather(1)
        _wait_gather(2)
        _wait_scatter(3)

        plsc.subcore_barrier()

    with jax.named_scope("emit_out"):
        _QR = _RPT // 4
        for k in range(4):
            pltpu.async_copy(acc_sh.at[pl.ds(s * _RPT + k * _QR, _QR)],
                             colmaj_v.at[pl.ds(k * _QR, _QR)], gsem[k])
        for k in range(4):
            pltpu.make_async_copy(acc_sh.at[pl.ds(s * _RPT, _QR)],
                                  colmaj_v.at[pl.ds(k * _QR, _QR)],
                                  gsem[k]).wait()

            @plsc.parallel_loop(k * _QR, (k + 1) * _QR, unroll=16)
            def _txpo(i):
                row = colmaj_v[i, :]
                plsc.store_scatter(rowmaj_v,
                                   [lanes, jnp.full((16,), i, jnp.int32)], row)
            pltpu.async_copy(rowmaj_v.at[:, pl.ds(k * _QR, _QR)],
                             out_hbm.at[c, :, pl.ds(s * _RPT + k * _QR, _QR)],
                             ssem[0])
        for k in range(4):
            pltpu.make_async_copy(rowmaj_v.at[:, pl.ds(0, _QR)],
                                  out_hbm.at[c, :, pl.ds(s * _RPT, _QR)],
                                  ssem[0]).wait()


_sc_call = functools.partial(
    pl.kernel,
    out_type=jax.ShapeDtypeStruct((_NC, _B, _M), jnp.float32),
    mesh=plsc.VectorSubcoreMesh(core_axis_name="c", subcore_axis_name="s"),
    scratch_types=[
        pltpu.VMEM((_NCHUNK, _CH), jnp.int32),
        pltpu.VMEM((_NCHUNK, _CH), jnp.int32),
        pltpu.VMEM((_NCHUNK, _CH), jnp.float32),
        pltpu.VMEM((_CH, _B), jnp.float32),
        pltpu.VMEM((_CH, _B), jnp.float32),
        pltpu.VMEM((_CH, _B), jnp.float32),
        pltpu.VMEM((_CH, _B), jnp.float32),
        pltpu.VMEM((_RPT, _B), jnp.float32),
        pltpu.VMEM((_B, _RPT), jnp.float32),
        pltpu.VMEM_SHARED((_N, _B), jnp.float32),
        pltpu.VMEM_SHARED((_M, _B), jnp.float32),
        pltpu.SemaphoreType.DMA,
        pltpu.SemaphoreType.DMA,
        pltpu.SemaphoreType.DMA,
        pltpu.SemaphoreType.DMA,
        pltpu.SemaphoreType.DMA,
        pltpu.SemaphoreType.DMA,
        pltpu.SemaphoreType.DMA,
        pltpu.SemaphoreType.DMA,
    ],
    compiler_params=pltpu.CompilerParams(use_tc_tiling_on_sc=False,
                                         needs_layout_passes=False),
)(_sc_body)


def _combine_body(p_ref, b_ref, o_ref):
    o_ref[...] = p_ref[0] + p_ref[1] + b_ref[...]


def _combine(partials, bias_row):
    return pl.pallas_call(
        _combine_body,
        in_specs=[
            pl.BlockSpec((_NC, _B, _M), lambda: (0, 0, 0)),
            pl.BlockSpec((1, _M), lambda: (0, 0)),
        ],
        out_specs=pl.BlockSpec((_B, _M), lambda: (0, 0)),
        out_shape=jax.ShapeDtypeStruct((_B, _M), jnp.float32),
    )(partials, bias_row)


def kernel(x, values, bias, edge_index):
    x2 = x.reshape(_B, _N)
    src2 = edge_index[0].reshape(_E // _CH, _CH)
    dst2 = edge_index[1].reshape(_E // _CH, _CH)
    val2 = values.reshape(_E // _CH, _CH)
    partials = _sc_call(x2, src2, dst2, val2)
    combined = _combine(partials, bias.reshape(1, _M))
    return combined[:, :, None]
